# Initial kernel scaffold; baseline (speedup 1.0000x reference)
import jax
import jax.numpy as jnp
from jax.experimental import pallas as pl


def kernel(x_cell, centroids_cell, x_tissue_3, centroids_tissue_3, assignment_matrix_3, params):
    raise NotImplementedError("write your pallas kernel here")


# speedup vs baseline: 1.1238x; 1.1238x over previous
import functools

import jax
import jax.numpy as jnp
from jax.experimental import pallas as pl

N_CELL = 10000
N_TISSUE = 1000
FDIM = 512
HID = 256
OUT = 256
LOC = 32
CLASSES = 7
NHEAD = 8
R = 10.0
K = 8


def _lin(x, w, b):
    return x @ w.T + b


def _lrelu(x):
    return jax.nn.leaky_relu(x, 0.01)


def _bn(x, g, b):
    m = x.mean(0, keepdims=True)
    v = x.var(0, keepdims=True)
    return (x - m) / jnp.sqrt(v + 1e-5) * g + b


def _graph_norm(x, g, b, a):
    m = x.mean(0, keepdims=True)
    o = x - a * m
    v = (o * o).mean(0, keepdims=True)
    return o / jnp.sqrt(v + 1e-5) * g + b


def _layer_norm(x, g, b):
    m = x.mean(-1, keepdims=True)
    v = x.var(-1, keepdims=True)
    return (x - m) / jnp.sqrt(v + 1e-5) * g + b


def _tower_kern(x_ref, w1_ref, b1_ref, w2_ref, b2_ref, o_ref):
    h = jnp.dot(x_ref[...], w1_ref[...].T, preferred_element_type=jnp.float32) + b1_ref[...]
    h = jax.nn.leaky_relu(h, 0.01)
    o_ref[...] = jnp.dot(h, w2_ref[...].T, preferred_element_type=jnp.float32) + b2_ref[...]


def _attr_tower(x, w1, b1, w2, b2, block=1000):
    n = x.shape[0]
    grid = (n // block,)
    return pl.pallas_call(
        _tower_kern,
        grid=grid,
        in_specs=[
            pl.BlockSpec((block, x.shape[1]), lambda i: (i, 0)),
            pl.BlockSpec(w1.shape, lambda i: (0, 0)),
            pl.BlockSpec(b1.shape, lambda i: (0,)),
            pl.BlockSpec(w2.shape, lambda i: (0, 0)),
            pl.BlockSpec(b2.shape, lambda i: (0,)),
        ],
        out_specs=pl.BlockSpec((block, w2.shape[0]), lambda i: (i, 0)),
        out_shape=jax.ShapeDtypeStruct((n, w2.shape[0]), jnp.float32),
    )(x, w1, b1, w2, b2)


def _radius_graph(feat, r, k):
    n = feat.shape[0]
    sq = jnp.sum(feat * feat, axis=1)
    d2 = sq[:, None] + sq[None, :] - 2.0 * (feat @ feat.T)
    d2 = jnp.maximum(d2, 0.0)
    vals, idx = jax.lax.top_k(-d2, k)
    d = -vals
    qidx = jnp.broadcast_to(jnp.arange(n)[:, None], (n, k))
    valid = (d <= r * r) & (idx != qidx)
    return idx.reshape(-1), qidx.reshape(-1), valid.reshape(-1).astype(jnp.float32)


def _sage(x, src, valid, wl, wr, b, n):
    f = x.shape[1]
    msgs = x[src].reshape(n, K, f) * valid.reshape(n, K, 1)
    s = msgs.sum(1)
    c = valid.reshape(n, K).sum(1)
    mean = s / jnp.maximum(c, 1.0)[:, None]
    return mean @ wl.T + x @ wr.T + b


def _mha(x, in_w, in_b, out_w, out_b, nhead):
    B, S, D = x.shape
    qkv = x @ in_w.T + in_b
    q, k, v = jnp.split(qkv, 3, axis=-1)
    dh = D // nhead

    def rs(t):
        return t.reshape(B, S, nhead, dh).transpose(0, 2, 1, 3)

    q, k, v = rs(q), rs(k), rs(v)
    att = jax.nn.softmax(q @ k.transpose(0, 1, 3, 2) / (float(dh) ** 0.5), axis=-1)
    o = (att @ v).transpose(0, 2, 1, 3).reshape(B, S, D)
    return o @ out_w.T + out_b


def _encoder_layer(x, p, l):
    a = _mha(x, p['t%d_in_w' % l], p['t%d_in_b' % l], p['t%d_out_w' % l], p['t%d_out_b' % l], NHEAD)
    x = _layer_norm(x + a, p['t%d_ln1_g' % l], p['t%d_ln1_b' % l])
    f = jax.nn.relu(_lin(x, p['t%d_ff1_w' % l], p['t%d_ff1_b' % l]))
    f = _lin(f, p['t%d_ff2_w' % l], p['t%d_ff2_b' % l])
    return _layer_norm(x + f, p['t%d_ln2_g' % l], p['t%d_ln2_b' % l])


def kernel(x_cell, centroids_cell, x_tissue_3, centroids_tissue_3, assignment_matrix_3, params):
    p = params
    batch_idx = jnp.argmax(assignment_matrix_3, axis=1)

    ca = _attr_tower(x_cell, p['ca1_w'], p['ca1_b'], p['ca2_w'], p['ca2_b'])
    cl = _bn(centroids_cell, p['cl_bn_g'], p['cl_bn_b'])
    cl = _lin(_lrelu(_lin(cl, p['cl1_w'], p['cl1_b'])), p['cl2_w'], p['cl2_b'])
    cell_feat = jnp.concatenate([ca, cl], axis=1)
    ta = _attr_tower(x_tissue_3, p['ta1_w'], p['ta1_b'], p['ta2_w'], p['ta2_b'])
    tl = _bn(centroids_tissue_3, p['tl_bn_g'], p['tl_bn_b'])
    tl = _lin(_lrelu(_lin(tl, p['tl1_w'], p['tl1_b'])), p['tl2_w'], p['tl2_b'])
    tis_feat = jnp.concatenate([ta, tl], axis=1)

    c_src, c_dst, c_val = _radius_graph(cell_feat, R, K)
    t_src, t_dst, t_val = _radius_graph(tis_feat, R, K)

    xc = x_cell
    for j in (1, 2, 3):
        xc = _sage(xc, c_src, c_val, p['c%d_wl' % j], p['c%d_wr' % j], p['c%d_b' % j], N_CELL)
        xc = _lrelu(_graph_norm(xc, p['gn_g'], p['gn_b'], p['gn_a']))
    xt = x_tissue_3
    for j in (4, 5, 6):
        xt = _sage(xt, t_src, t_val, p['c%d_wl' % j], p['c%d_wr' % j], p['c%d_b' % j], N_TISSUE)
        xt = _lrelu(_graph_norm(xt, p['gn_g'], p['gn_b'], p['gn_a']))

    xt_for_cell = xt[batch_idx][:, None, :]
    seq = jnp.concatenate(
        [jnp.broadcast_to(p['cls'], (N_CELL, 1, OUT)), xc[:, None, :], xt_for_cell], axis=1)
    seq = seq + p['pos'][:, :3]
    for l in range(2):
        seq = _encoder_layer(seq, p, l)
    feat = seq[:, 0]
    att = _lin(_lrelu(_lin(feat, p['att1_w'], p['att1_b'])), p['att2_w'], p['att2_b'])
    att = jax.nn.softmax(att, axis=0)
    pooled = feat.mean(axis=0, keepdims=True)
    h = _lrelu(_lin(pooled, p['lin1_w'], p['lin1_b']))
    h = _layer_norm(h, p['ln2_g'], p['ln2_b'])
    logits = _lin(h, p['lin2_w'], p['lin2_b'])
    edge_index_cell = jnp.stack([c_src, c_dst])
    return logits, edge_index_cell, att


# speedup vs baseline: 2.2383x; 1.9918x over previous
import functools

import jax
import jax.numpy as jnp
from jax.experimental import pallas as pl

N_CELL = 10000
N_TISSUE = 1000
FDIM = 512
HID = 256
OUT = 256
LOC = 32
CLASSES = 7
NHEAD = 8
R = 10.0
K = 8


def _lin(x, w, b):
    return x @ w.T + b


def _lrelu(x):
    return jax.nn.leaky_relu(x, 0.01)


def _bn(x, g, b):
    m = x.mean(0, keepdims=True)
    v = x.var(0, keepdims=True)
    return (x - m) / jnp.sqrt(v + 1e-5) * g + b


def _graph_norm(x, g, b, a):
    m = x.mean(0, keepdims=True)
    o = x - a * m
    v = (o * o).mean(0, keepdims=True)
    return o / jnp.sqrt(v + 1e-5) * g + b


def _layer_norm(x, g, b):
    m = x.mean(-1, keepdims=True)
    v = x.var(-1, keepdims=True)
    return (x - m) / jnp.sqrt(v + 1e-5) * g + b


def _bdot(a, b):
    return jax.lax.dot_general(
        a.astype(jnp.bfloat16), b.astype(jnp.bfloat16),
        (((1,), (1,)), ((), ())), preferred_element_type=jnp.float32)


def _bdot_acc(a, b, csz=256):
    kdim = a.shape[1]
    if kdim <= csz:
        return _bdot(a, b)
    acc = _bdot(a[:, :csz], b[:, :csz])
    for c in range(csz, kdim, csz):
        acc = acc + _bdot(a[:, c:c + csz], b[:, c:c + csz])
    return acc


def _tower_kern(x_ref, w1_ref, b1_ref, w2_ref, b2_ref, o_ref):
    h = _bdot_acc(x_ref[...], w1_ref[...]) + b1_ref[...]
    h = jax.nn.leaky_relu(h, 0.01)
    o_ref[...] = _bdot_acc(h, w2_ref[...]) + b2_ref[...]


def _attr_tower(x, w1, b1, w2, b2, block=1000):
    n = x.shape[0]
    grid = (n // block,)
    return pl.pallas_call(
        _tower_kern,
        grid=grid,
        in_specs=[
            pl.BlockSpec((block, x.shape[1]), lambda i: (i, 0)),
            pl.BlockSpec(w1.shape, lambda i: (0, 0)),
            pl.BlockSpec(b1.shape, lambda i: (0,)),
            pl.BlockSpec(w2.shape, lambda i: (0, 0)),
            pl.BlockSpec(b2.shape, lambda i: (0,)),
        ],
        out_specs=pl.BlockSpec((block, w2.shape[0]), lambda i: (i, 0)),
        out_shape=jax.ShapeDtypeStruct((n, w2.shape[0]), jnp.float32),
    )(x, w1, b1, w2, b2)


_INT_MAX = 2**31 - 1


def _topk_kern(n, n_tiles, rb, ct, r2, featb_ref, feat_ref, sqb_ref, sq_ref,
               idx_ref, valid_ref):
    i = pl.program_id(0)
    q = featb_ref[...]
    sq_q = sqb_ref[...]

    def body(t, carry):
        run_v, run_i = carry
        g = feat_ref[pl.ds(pl.multiple_of(t * ct, ct), ct), :]
        sq_g = sq_ref[:, pl.ds(pl.multiple_of(t * ct, ct), ct)]
        qg = _bdot(q, g)
        d2 = jnp.maximum((sq_q + sq_g) - 2.0 * qg, 0.0)
        col = t * ct + jax.lax.broadcasted_iota(jnp.int32, (1, ct), 1)
        d2 = jnp.where(col < n, d2, jnp.inf)
        cv = jnp.concatenate([run_v, d2], axis=1)
        ci = jnp.concatenate([run_i, jnp.broadcast_to(col, d2.shape)], axis=1)
        nv, ni = [], []
        for _ in range(K):
            m = jnp.min(cv, axis=1, keepdims=True)
            sel = cv == m
            am = jnp.min(jnp.where(sel, ci, _INT_MAX), axis=1, keepdims=True)
            nv.append(m)
            ni.append(am)
            cv = jnp.where(sel & (ci == am), jnp.inf, cv)
        return jnp.concatenate(nv, axis=1), jnp.concatenate(ni, axis=1)

    run_v = jnp.full((rb, K), jnp.inf, jnp.float32)
    run_i = jnp.zeros((rb, K), jnp.int32)
    run_v, run_i = jax.lax.fori_loop(0, n_tiles, body, (run_v, run_i))
    rows = i * rb + jax.lax.broadcasted_iota(jnp.int32, (rb, 1), 0)
    idx_ref[...] = run_i
    valid_ref[...] = ((run_v <= r2) & (run_i != rows)).astype(jnp.float32)


def _radius_topk(feat, r, rb, ct):
    n, f = feat.shape
    n_tiles = -(-n // ct)
    n_pad = n_tiles * ct
    sq = jnp.sum(feat * feat, axis=1)
    featp = jnp.concatenate([feat, jnp.zeros((n_pad - n, f), feat.dtype)]) if n_pad > n else feat
    sqp = jnp.concatenate([sq, jnp.zeros((n_pad - n,), sq.dtype)]) if n_pad > n else sq
    kern = functools.partial(_topk_kern, n, n_tiles, rb, ct, r * r)
    return pl.pallas_call(
        kern,
        grid=(n // rb,),
        in_specs=[
            pl.BlockSpec((rb, f), lambda i: (i, 0)),
            pl.BlockSpec((n_pad, f), lambda i: (0, 0)),
            pl.BlockSpec((rb, 1), lambda i: (i, 0)),
            pl.BlockSpec((1, n_pad), lambda i: (0, 0)),
        ],
        out_specs=[
            pl.BlockSpec((rb, K), lambda i: (i, 0)),
            pl.BlockSpec((rb, K), lambda i: (i, 0)),
        ],
        out_shape=[
            jax.ShapeDtypeStruct((n, K), jnp.int32),
            jax.ShapeDtypeStruct((n, K), jnp.float32),
        ],
    )(feat, featp, sq[:, None], sqp[None, :])


def _sage(x, idx, valid, wl, wr, b, n):
    f = x.shape[1]
    msgs = x[idx.reshape(-1)].reshape(n, K, f) * valid[:, :, None]
    s = msgs.sum(1)
    c = valid.sum(1)
    mean = s / jnp.maximum(c, 1.0)[:, None]
    return mean @ wl.T + x @ wr.T + b


def _mha(x, in_w, in_b, out_w, out_b, nhead):
    B, S, D = x.shape
    qkv = x @ in_w.T + in_b
    q, k, v = jnp.split(qkv, 3, axis=-1)
    dh = D // nhead

    def rs(t):
        return t.reshape(B, S, nhead, dh).transpose(0, 2, 1, 3)

    q, k, v = rs(q), rs(k), rs(v)
    att = jax.nn.softmax(q @ k.transpose(0, 1, 3, 2) / (float(dh) ** 0.5), axis=-1)
    o = (att @ v).transpose(0, 2, 1, 3).reshape(B, S, D)
    return o @ out_w.T + out_b


def _encoder_layer(x, p, l):
    a = _mha(x, p['t%d_in_w' % l], p['t%d_in_b' % l], p['t%d_out_w' % l], p['t%d_out_b' % l], NHEAD)
    x = _layer_norm(x + a, p['t%d_ln1_g' % l], p['t%d_ln1_b' % l])
    f = jax.nn.relu(_lin(x, p['t%d_ff1_w' % l], p['t%d_ff1_b' % l]))
    f = _lin(f, p['t%d_ff2_w' % l], p['t%d_ff2_b' % l])
    return _layer_norm(x + f, p['t%d_ln2_g' % l], p['t%d_ln2_b' % l])


def kernel(x_cell, centroids_cell, x_tissue_3, centroids_tissue_3, assignment_matrix_3, params):
    p = params
    batch_idx = jnp.argmax(assignment_matrix_3, axis=1)

    ca = _attr_tower(x_cell, p['ca1_w'], p['ca1_b'], p['ca2_w'], p['ca2_b'])
    cl = _bn(centroids_cell, p['cl_bn_g'], p['cl_bn_b'])
    cl = _lin(_lrelu(_lin(cl, p['cl1_w'], p['cl1_b'])), p['cl2_w'], p['cl2_b'])
    cell_feat = jnp.concatenate([ca, cl], axis=1)
    ta = _attr_tower(x_tissue_3, p['ta1_w'], p['ta1_b'], p['ta2_w'], p['ta2_b'])
    tl = _bn(centroids_tissue_3, p['tl_bn_g'], p['tl_bn_b'])
    tl = _lin(_lrelu(_lin(tl, p['tl1_w'], p['tl1_b'])), p['tl2_w'], p['tl2_b'])
    tis_feat = jnp.concatenate([ta, tl], axis=1)

    c_idx, c_valid = _radius_topk(cell_feat, R, rb=1000, ct=512)
    t_idx, t_valid = _radius_topk(tis_feat, R, rb=1000, ct=512)

    xc = x_cell
    for j in (1, 2, 3):
        xc = _sage(xc, c_idx, c_valid, p['c%d_wl' % j], p['c%d_wr' % j], p['c%d_b' % j], N_CELL)
        xc = _lrelu(_graph_norm(xc, p['gn_g'], p['gn_b'], p['gn_a']))
    xt = x_tissue_3
    for j in (4, 5, 6):
        xt = _sage(xt, t_idx, t_valid, p['c%d_wl' % j], p['c%d_wr' % j], p['c%d_b' % j], N_TISSUE)
        xt = _lrelu(_graph_norm(xt, p['gn_g'], p['gn_b'], p['gn_a']))

    xt_for_cell = xt[batch_idx][:, None, :]
    seq = jnp.concatenate(
        [jnp.broadcast_to(p['cls'], (N_CELL, 1, OUT)), xc[:, None, :], xt_for_cell], axis=1)
    seq = seq + p['pos'][:, :3]
    for l in range(2):
        seq = _encoder_layer(seq, p, l)
    feat = seq[:, 0]
    att = _lin(_lrelu(_lin(feat, p['att1_w'], p['att1_b'])), p['att2_w'], p['att2_b'])
    att = jax.nn.softmax(att, axis=0)
    pooled = feat.mean(axis=0, keepdims=True)
    h = _lrelu(_lin(pooled, p['lin1_w'], p['lin1_b']))
    h = _layer_norm(h, p['ln2_g'], p['ln2_b'])
    logits = _lin(h, p['lin2_w'], p['lin2_b'])
    c_src = c_idx.reshape(-1)
    c_dst = jnp.broadcast_to(
        jnp.arange(N_CELL, dtype=c_idx.dtype)[:, None], (N_CELL, K)).reshape(-1)
    edge_index_cell = jnp.stack([c_src, c_dst])
    return logits, edge_index_cell, att


# speedup vs baseline: 3.8611x; 1.7250x over previous
import functools

import jax
import jax.numpy as jnp
from jax.experimental import pallas as pl

N_CELL = 10000
N_TISSUE = 1000
FDIM = 512
HID = 256
OUT = 256
LOC = 32
CLASSES = 7
NHEAD = 8
R = 10.0
K = 8


def _lin(x, w, b):
    return x @ w.T + b


def _lrelu(x):
    return jax.nn.leaky_relu(x, 0.01)


def _bn(x, g, b):
    m = x.mean(0, keepdims=True)
    v = x.var(0, keepdims=True)
    return (x - m) / jnp.sqrt(v + 1e-5) * g + b


def _graph_norm(x, g, b, a):
    m = x.mean(0, keepdims=True)
    o = x - a * m
    v = (o * o).mean(0, keepdims=True)
    return o / jnp.sqrt(v + 1e-5) * g + b


def _layer_norm(x, g, b):
    m = x.mean(-1, keepdims=True)
    v = x.var(-1, keepdims=True)
    return (x - m) / jnp.sqrt(v + 1e-5) * g + b


def _bdot(a, b):
    return jax.lax.dot_general(
        a.astype(jnp.bfloat16), b.astype(jnp.bfloat16),
        (((1,), (1,)), ((), ())), preferred_element_type=jnp.float32)


def _bdot_acc(a, b, csz=256):
    kdim = a.shape[1]
    if kdim <= csz:
        return _bdot(a, b)
    acc = _bdot(a[:, :csz], b[:, :csz])
    for c in range(csz, kdim, csz):
        acc = acc + _bdot(a[:, c:c + csz], b[:, c:c + csz])
    return acc


def _tower_kern(x_ref, w1_ref, b1_ref, w2_ref, b2_ref, o_ref):
    h = _bdot_acc(x_ref[...], w1_ref[...]) + b1_ref[...]
    h = jax.nn.leaky_relu(h, 0.01)
    o_ref[...] = _bdot_acc(h, w2_ref[...]) + b2_ref[...]


def _attr_tower(x, w1, b1, w2, b2, block=1000):
    n = x.shape[0]
    grid = (n // block,)
    return pl.pallas_call(
        _tower_kern,
        grid=grid,
        in_specs=[
            pl.BlockSpec((block, x.shape[1]), lambda i: (i, 0)),
            pl.BlockSpec(w1.shape, lambda i: (0, 0)),
            pl.BlockSpec(b1.shape, lambda i: (0,)),
            pl.BlockSpec(w2.shape, lambda i: (0, 0)),
            pl.BlockSpec(b2.shape, lambda i: (0,)),
        ],
        out_specs=pl.BlockSpec((block, w2.shape[0]), lambda i: (i, 0)),
        out_shape=jax.ShapeDtypeStruct((n, w2.shape[0]), jnp.float32),
    )(x, w1, b1, w2, b2)


_INT_MAX = 2**31 - 1


def _topk_kern(n, n_tiles, rb, ct, r2, featb_ref, feat_ref, sqb_ref, sq_ref,
               idx_ref, valid_ref):
    i = pl.program_id(0)
    q = featb_ref[...]
    sq_q = sqb_ref[...]

    def body(t, carry):
        run_v, run_i = carry
        g = feat_ref[pl.ds(pl.multiple_of(t * ct, ct), ct), :]
        sq_g = sq_ref[:, pl.ds(pl.multiple_of(t * ct, ct), ct)]
        qg = _bdot(q, g)
        d2 = jnp.maximum((sq_q + sq_g) - 2.0 * qg, 0.0)
        col = t * ct + jax.lax.broadcasted_iota(jnp.int32, (1, ct), 1)
        d2 = jnp.where(col < n, d2, jnp.inf)
        cv = jnp.concatenate([run_v, d2], axis=1)
        ci = jnp.concatenate([run_i, jnp.broadcast_to(col, d2.shape)], axis=1)
        nv, ni = [], []
        for _ in range(K):
            m = jnp.min(cv, axis=1, keepdims=True)
            sel = cv == m
            am = jnp.min(jnp.where(sel, ci, _INT_MAX), axis=1, keepdims=True)
            nv.append(m)
            ni.append(am)
            cv = jnp.where(sel & (ci == am), jnp.inf, cv)
        return jnp.concatenate(nv, axis=1), jnp.concatenate(ni, axis=1)

    run_v = jnp.full((rb, K), jnp.inf, jnp.float32)
    run_i = jnp.zeros((rb, K), jnp.int32)
    run_v, run_i = jax.lax.fori_loop(0, n_tiles, body, (run_v, run_i))
    rows = i * rb + jax.lax.broadcasted_iota(jnp.int32, (rb, 1), 0)
    idx_ref[...] = run_i
    valid_ref[...] = ((run_v <= r2) & (run_i != rows)).astype(jnp.float32)


def _radius_topk(feat, r, rb, ct):
    n, f = feat.shape
    n_tiles = -(-n // ct)
    n_pad = n_tiles * ct
    sq = jnp.sum(feat * feat, axis=1)
    featp = jnp.concatenate([feat, jnp.zeros((n_pad - n, f), feat.dtype)]) if n_pad > n else feat
    sqp = jnp.concatenate([sq, jnp.zeros((n_pad - n,), sq.dtype)]) if n_pad > n else sq
    kern = functools.partial(_topk_kern, n, n_tiles, rb, ct, r * r)
    return pl.pallas_call(
        kern,
        grid=(n // rb,),
        in_specs=[
            pl.BlockSpec((rb, f), lambda i: (i, 0)),
            pl.BlockSpec((n_pad, f), lambda i: (0, 0)),
            pl.BlockSpec((rb, 1), lambda i: (i, 0)),
            pl.BlockSpec((1, n_pad), lambda i: (0, 0)),
        ],
        out_specs=[
            pl.BlockSpec((rb, K), lambda i: (i, 0)),
            pl.BlockSpec((rb, K), lambda i: (i, 0)),
        ],
        out_shape=[
            jax.ShapeDtypeStruct((n, K), jnp.int32),
            jax.ShapeDtypeStruct((n, K), jnp.float32),
        ],
    )(feat, featp, sq[:, None], sqp[None, :])


def _sage(x, idx, valid, wl, wr, b, n):
    f = x.shape[1]
    msgs = x[idx.reshape(-1)].reshape(n, K, f) * valid[:, :, None]
    s = msgs.sum(1)
    c = valid.sum(1)
    mean = s / jnp.maximum(c, 1.0)[:, None]
    return mean @ wl.T + x @ wr.T + b


_DH = OUT // NHEAD
_SCALE = float(_DH) ** 0.5

_ENC_PNAMES = []
for _l in range(2):
    _ENC_PNAMES += ['t%d_%s' % (_l, s) for s in
                    ('in_w', 'in_b', 'out_w', 'out_b', 'ln1_g', 'ln1_b',
                     'ff1_w', 'ff1_b', 'ff2_w', 'ff2_b', 'ln2_g', 'ln2_b')]
_ENC_PNAMES += ['att1_w', 'att1_b', 'att2_w', 'att2_b']


def _ln_rows(x, g, b):
    m = jnp.mean(x, axis=1, keepdims=True)
    d = x - m
    v = jnp.mean(d * d, axis=1, keepdims=True)
    return d * jax.lax.rsqrt(v + 1e-5) * g + b


def _enc_kern(cls_ref, pos_ref, xc_ref, xt_ref, *refs):
    pr = {n: r[...] for n, r in zip(_ENC_PNAMES, refs[:len(_ENC_PNAMES)])}
    att_ref, pool_ref = refs[len(_ENC_PNAMES):]
    rb = xc_ref.shape[0]
    lane = jax.lax.broadcasted_iota(jnp.int32, (OUT, NHEAD), 0)
    head = jax.lax.broadcasted_iota(jnp.int32, (OUT, NHEAD), 1)
    G = (lane // _DH == head).astype(jnp.float32)

    t = [jnp.broadcast_to(cls_ref[...] + pos_ref[0:1, :], (rb, OUT)),
         xc_ref[...] + pos_ref[1:2, :],
         xt_ref[...] + pos_ref[2:3, :]]

    for l in range(2):
        in_w = pr['t%d_in_w' % l]
        wq, wk, wv = in_w[:OUT], in_w[OUT:2 * OUT], in_w[2 * OUT:]
        in_b = pr['t%d_in_b' % l]
        bq, bk, bv = in_b[:OUT], in_b[OUT:2 * OUT], in_b[2 * OUT:]
        q = [_bdot(ti, wq) + bq for ti in t]
        k = [_bdot(ti, wk) + bk for ti in t]
        v = [_bdot(ti, wv) + bv for ti in t]
        logit = [[jax.lax.dot_general(
            q[i] * k[j], G, (((1,), (0,)), ((), ())),
            preferred_element_type=jnp.float32) / _SCALE
            for j in range(3)] for i in range(3)]
        a = []
        for i in range(3):
            m = jnp.maximum(jnp.maximum(logit[i][0], logit[i][1]), logit[i][2])
            e = [jnp.exp(logit[i][j] - m) for j in range(3)]
            tot = e[0] + e[1] + e[2]
            o = None
            for j in range(3):
                s_full = jax.lax.dot_general(
                    e[j] / tot, G, (((1,), (1,)), ((), ())),
                    preferred_element_type=jnp.float32)
                term = s_full * v[j]
                o = term if o is None else o + term
            a.append(_bdot(o, pr['t%d_out_w' % l]) + pr['t%d_out_b' % l])
        x = [_ln_rows(t[i] + a[i], pr['t%d_ln1_g' % l], pr['t%d_ln1_b' % l])
             for i in range(3)]
        t = []
        for i in range(3):
            f = jnp.maximum(_bdot(x[i], pr['t%d_ff1_w' % l]) + pr['t%d_ff1_b' % l], 0.0)
            f = _bdot(f, pr['t%d_ff2_w' % l]) + pr['t%d_ff2_b' % l]
            t.append(_ln_rows(x[i] + f, pr['t%d_ln2_g' % l], pr['t%d_ln2_b' % l]))

    feat = t[0]
    h = jax.nn.leaky_relu(_bdot(feat, pr['att1_w']) + pr['att1_b'], 0.01)
    att_ref[...] = jnp.sum(h * pr['att2_w'], axis=1, keepdims=True) + pr['att2_b']
    pool_ref[...] = jnp.sum(feat, axis=0, keepdims=True)[None]


def _encoder_head(xc, xtg, p, rb=1000):
    n = xc.shape[0]
    grid = (n // rb,)
    enc_params = [p[name] for name in _ENC_PNAMES]
    cls2 = p['cls'].reshape(1, OUT)
    pos2 = p['pos'].reshape(3, OUT)
    full = lambda a: pl.BlockSpec(a.shape, lambda i: (0,) * a.ndim)
    att_raw, pools = pl.pallas_call(
        _enc_kern,
        grid=grid,
        in_specs=[
            full(cls2),
            full(pos2),
            pl.BlockSpec((rb, OUT), lambda i: (i, 0)),
            pl.BlockSpec((rb, OUT), lambda i: (i, 0)),
        ] + [full(a) for a in enc_params],
        out_specs=[
            pl.BlockSpec((rb, 1), lambda i: (i, 0)),
            pl.BlockSpec((1, 1, OUT), lambda i: (i, 0, 0)),
        ],
        out_shape=[
            jax.ShapeDtypeStruct((n, 1), jnp.float32),
            jax.ShapeDtypeStruct((grid[0], 1, OUT), jnp.float32),
        ],
    )(cls2, pos2, xc, xtg, *enc_params)
    return att_raw, pools.reshape(grid[0], OUT)


def kernel(x_cell, centroids_cell, x_tissue_3, centroids_tissue_3, assignment_matrix_3, params):
    p = params
    batch_idx = jnp.argmax(assignment_matrix_3, axis=1)

    ca = _attr_tower(x_cell, p['ca1_w'], p['ca1_b'], p['ca2_w'], p['ca2_b'])
    cl = _bn(centroids_cell, p['cl_bn_g'], p['cl_bn_b'])
    cl = _lin(_lrelu(_lin(cl, p['cl1_w'], p['cl1_b'])), p['cl2_w'], p['cl2_b'])
    cell_feat = jnp.concatenate([ca, cl], axis=1)
    ta = _attr_tower(x_tissue_3, p['ta1_w'], p['ta1_b'], p['ta2_w'], p['ta2_b'])
    tl = _bn(centroids_tissue_3, p['tl_bn_g'], p['tl_bn_b'])
    tl = _lin(_lrelu(_lin(tl, p['tl1_w'], p['tl1_b'])), p['tl2_w'], p['tl2_b'])
    tis_feat = jnp.concatenate([ta, tl], axis=1)

    c_idx, c_valid = _radius_topk(cell_feat, R, rb=1000, ct=512)
    t_idx, t_valid = _radius_topk(tis_feat, R, rb=1000, ct=512)

    xc = x_cell
    for j in (1, 2, 3):
        xc = _sage(xc, c_idx, c_valid, p['c%d_wl' % j], p['c%d_wr' % j], p['c%d_b' % j], N_CELL)
        xc = _lrelu(_graph_norm(xc, p['gn_g'], p['gn_b'], p['gn_a']))
    xt = x_tissue_3
    for j in (4, 5, 6):
        xt = _sage(xt, t_idx, t_valid, p['c%d_wl' % j], p['c%d_wr' % j], p['c%d_b' % j], N_TISSUE)
        xt = _lrelu(_graph_norm(xt, p['gn_g'], p['gn_b'], p['gn_a']))

    xtg = xt[batch_idx]
    att_raw, pools = _encoder_head(xc, xtg, p)
    att = jax.nn.softmax(att_raw, axis=0)
    pooled = jnp.sum(pools, axis=0, keepdims=True) * (1.0 / N_CELL)
    h = _lrelu(_lin(pooled, p['lin1_w'], p['lin1_b']))
    h = _layer_norm(h, p['ln2_g'], p['ln2_b'])
    logits = _lin(h, p['lin2_w'], p['lin2_b'])
    c_src = c_idx.reshape(-1)
    c_dst = jnp.broadcast_to(
        jnp.arange(N_CELL, dtype=c_idx.dtype)[:, None], (N_CELL, K)).reshape(-1)
    edge_index_cell = jnp.stack([c_src, c_dst])
    return logits, edge_index_cell, att


# speedup vs baseline: 3.9885x; 1.0330x over previous
import functools

import jax
import jax.numpy as jnp
from jax.experimental import pallas as pl

N_CELL = 10000
N_TISSUE = 1000
FDIM = 512
HID = 256
OUT = 256
LOC = 32
CLASSES = 7
NHEAD = 8
R = 10.0
K = 8


def _lin(x, w, b):
    return x @ w.T + b


def _lrelu(x):
    return jax.nn.leaky_relu(x, 0.01)


def _bn(x, g, b):
    m = x.mean(0, keepdims=True)
    v = x.var(0, keepdims=True)
    return (x - m) / jnp.sqrt(v + 1e-5) * g + b


def _graph_norm(x, g, b, a):
    m = x.mean(0, keepdims=True)
    o = x - a * m
    v = (o * o).mean(0, keepdims=True)
    return o / jnp.sqrt(v + 1e-5) * g + b


def _layer_norm(x, g, b):
    m = x.mean(-1, keepdims=True)
    v = x.var(-1, keepdims=True)
    return (x - m) / jnp.sqrt(v + 1e-5) * g + b


def _bdot(a, b):
    return jax.lax.dot_general(
        a.astype(jnp.bfloat16), b.astype(jnp.bfloat16),
        (((1,), (1,)), ((), ())), preferred_element_type=jnp.float32)


def _bdot_acc(a, b, csz=256):
    kdim = a.shape[1]
    if kdim <= csz:
        return _bdot(a, b)
    acc = _bdot(a[:, :csz], b[:, :csz])
    for c in range(csz, kdim, csz):
        acc = acc + _bdot(a[:, c:c + csz], b[:, c:c + csz])
    return acc


def _tower_kern(x_ref, w1_ref, b1_ref, w2_ref, b2_ref, o_ref):
    h = _bdot_acc(x_ref[...], w1_ref[...]) + b1_ref[...]
    h = jax.nn.leaky_relu(h, 0.01)
    o_ref[...] = _bdot_acc(h, w2_ref[...]) + b2_ref[...]


def _attr_tower(x, w1, b1, w2, b2, block=1000):
    n = x.shape[0]
    grid = (n // block,)
    return pl.pallas_call(
        _tower_kern,
        grid=grid,
        in_specs=[
            pl.BlockSpec((block, x.shape[1]), lambda i: (i, 0)),
            pl.BlockSpec(w1.shape, lambda i: (0, 0)),
            pl.BlockSpec(b1.shape, lambda i: (0,)),
            pl.BlockSpec(w2.shape, lambda i: (0, 0)),
            pl.BlockSpec(b2.shape, lambda i: (0,)),
        ],
        out_specs=pl.BlockSpec((block, w2.shape[0]), lambda i: (i, 0)),
        out_shape=jax.ShapeDtypeStruct((n, w2.shape[0]), jnp.float32),
    )(x, w1, b1, w2, b2)


_INT_MAX = 2**31 - 1


def _topk_kern(n, n_tiles, rb, ct, r2, featb_ref, feat_ref, sqb_ref, sq_ref,
               idx_ref, valid_ref):
    i = pl.program_id(0)
    q = featb_ref[...]
    sq_q = sqb_ref[...]

    def body(t, carry):
        run_v, run_i = carry
        g = feat_ref[pl.ds(pl.multiple_of(t * ct, ct), ct), :]
        sq_g = sq_ref[:, pl.ds(pl.multiple_of(t * ct, ct), ct)]
        qg = _bdot(q, g)
        d2 = jnp.maximum((sq_q + sq_g) - 2.0 * qg, 0.0)
        col = t * ct + jax.lax.broadcasted_iota(jnp.int32, (1, ct), 1)
        d2 = jnp.where(col < n, d2, jnp.inf)
        cv = jnp.concatenate([run_v, d2], axis=1)
        ci = jnp.concatenate([run_i, jnp.broadcast_to(col, d2.shape)], axis=1)
        nv, ni = [], []
        for _ in range(K):
            m = jnp.min(cv, axis=1, keepdims=True)
            sel = cv == m
            am = jnp.min(jnp.where(sel, ci, _INT_MAX), axis=1, keepdims=True)
            nv.append(m)
            ni.append(am)
            cv = jnp.where(sel & (ci == am), jnp.inf, cv)
        return jnp.concatenate(nv, axis=1), jnp.concatenate(ni, axis=1)

    run_v = jnp.full((rb, K), jnp.inf, jnp.float32)
    run_i = jnp.zeros((rb, K), jnp.int32)
    run_v, run_i = jax.lax.fori_loop(0, n_tiles, body, (run_v, run_i))
    rows = i * rb + jax.lax.broadcasted_iota(jnp.int32, (rb, 1), 0)
    idx_ref[...] = run_i
    valid_ref[...] = ((run_v <= r2) & (run_i != rows)).astype(jnp.float32)


def _radius_topk(feat, r, rb, ct):
    n, f = feat.shape
    n_tiles = -(-n // ct)
    n_pad = n_tiles * ct
    sq = jnp.sum(feat * feat, axis=1)
    featp = jnp.concatenate([feat, jnp.zeros((n_pad - n, f), feat.dtype)]) if n_pad > n else feat
    sqp = jnp.concatenate([sq, jnp.zeros((n_pad - n,), sq.dtype)]) if n_pad > n else sq
    kern = functools.partial(_topk_kern, n, n_tiles, rb, ct, r * r)
    return pl.pallas_call(
        kern,
        grid=(n // rb,),
        in_specs=[
            pl.BlockSpec((rb, f), lambda i: (i, 0)),
            pl.BlockSpec((n_pad, f), lambda i: (0, 0)),
            pl.BlockSpec((rb, 1), lambda i: (i, 0)),
            pl.BlockSpec((1, n_pad), lambda i: (0, 0)),
        ],
        out_specs=[
            pl.BlockSpec((rb, K), lambda i: (i, 0)),
            pl.BlockSpec((rb, K), lambda i: (i, 0)),
        ],
        out_shape=[
            jax.ShapeDtypeStruct((n, K), jnp.int32),
            jax.ShapeDtypeStruct((n, K), jnp.float32),
        ],
    )(feat, featp, sq[:, None], sqp[None, :])


def _sage(x, idx, valid, wl, wr, b, n):
    z = x @ wl.T
    msgs = z[idx.reshape(-1)].reshape(n, K, z.shape[1]) * valid[:, :, None]
    s = msgs.sum(1)
    c = valid.sum(1)
    mean = s / jnp.maximum(c, 1.0)[:, None]
    return mean + x @ wr.T + b


_DH = OUT // NHEAD
_SCALE = float(_DH) ** 0.5

_ENC_PNAMES = []
for _l in range(2):
    _ENC_PNAMES += ['t%d_%s' % (_l, s) for s in
                    ('in_w', 'in_b', 'out_w', 'out_b', 'ln1_g', 'ln1_b',
                     'ff1_w', 'ff1_b', 'ff2_w', 'ff2_b', 'ln2_g', 'ln2_b')]
_ENC_PNAMES += ['att1_w', 'att1_b', 'att2_w', 'att2_b']


def _ln_rows(x, g, b):
    m = jnp.mean(x, axis=1, keepdims=True)
    d = x - m
    v = jnp.mean(d * d, axis=1, keepdims=True)
    return d * jax.lax.rsqrt(v + 1e-5) * g + b


def _enc_kern(cls_ref, pos_ref, xc_ref, xt_ref, *refs):
    pr = {n: r[...] for n, r in zip(_ENC_PNAMES, refs[:len(_ENC_PNAMES)])}
    att_ref, pool_ref = refs[len(_ENC_PNAMES):]
    rb = xc_ref.shape[0]
    lane = jax.lax.broadcasted_iota(jnp.int32, (OUT, NHEAD), 0)
    head = jax.lax.broadcasted_iota(jnp.int32, (OUT, NHEAD), 1)
    G = (lane // _DH == head).astype(jnp.float32)

    t = [jnp.broadcast_to(cls_ref[...] + pos_ref[0:1, :], (rb, OUT)),
         xc_ref[...] + pos_ref[1:2, :],
         xt_ref[...] + pos_ref[2:3, :]]

    for l in range(2):
        in_w = pr['t%d_in_w' % l]
        wq, wk, wv = in_w[:OUT], in_w[OUT:2 * OUT], in_w[2 * OUT:]
        in_b = pr['t%d_in_b' % l]
        bq, bk, bv = in_b[:OUT], in_b[OUT:2 * OUT], in_b[2 * OUT:]
        q = [_bdot(ti, wq) + bq for ti in t]
        k = [_bdot(ti, wk) + bk for ti in t]
        v = [_bdot(ti, wv) + bv for ti in t]
        logit = [[jax.lax.dot_general(
            q[i] * k[j], G, (((1,), (0,)), ((), ())),
            preferred_element_type=jnp.float32) / _SCALE
            for j in range(3)] for i in range(3)]
        a = []
        for i in range(3):
            m = jnp.maximum(jnp.maximum(logit[i][0], logit[i][1]), logit[i][2])
            e = [jnp.exp(logit[i][j] - m) for j in range(3)]
            tot = e[0] + e[1] + e[2]
            o = None
            for j in range(3):
                s_full = jax.lax.dot_general(
                    e[j] / tot, G, (((1,), (1,)), ((), ())),
                    preferred_element_type=jnp.float32)
                term = s_full * v[j]
                o = term if o is None else o + term
            a.append(_bdot(o, pr['t%d_out_w' % l]) + pr['t%d_out_b' % l])
        x = [_ln_rows(t[i] + a[i], pr['t%d_ln1_g' % l], pr['t%d_ln1_b' % l])
             for i in range(3)]
        t = []
        for i in range(3):
            f = jnp.maximum(_bdot(x[i], pr['t%d_ff1_w' % l]) + pr['t%d_ff1_b' % l], 0.0)
            f = _bdot(f, pr['t%d_ff2_w' % l]) + pr['t%d_ff2_b' % l]
            t.append(_ln_rows(x[i] + f, pr['t%d_ln2_g' % l], pr['t%d_ln2_b' % l]))

    feat = t[0]
    h = jax.nn.leaky_relu(_bdot(feat, pr['att1_w']) + pr['att1_b'], 0.01)
    att_ref[...] = jnp.sum(h * pr['att2_w'], axis=1, keepdims=True) + pr['att2_b']
    pool_ref[...] = jnp.sum(feat, axis=0, keepdims=True)[None]


def _encoder_head(xc, xtg, p, rb=1000):
    n = xc.shape[0]
    grid = (n // rb,)
    enc_params = [p[name] for name in _ENC_PNAMES]
    cls2 = p['cls'].reshape(1, OUT)
    pos2 = p['pos'].reshape(3, OUT)
    full = lambda a: pl.BlockSpec(a.shape, lambda i: (0,) * a.ndim)
    att_raw, pools = pl.pallas_call(
        _enc_kern,
        grid=grid,
        in_specs=[
            full(cls2),
            full(pos2),
            pl.BlockSpec((rb, OUT), lambda i: (i, 0)),
            pl.BlockSpec((rb, OUT), lambda i: (i, 0)),
        ] + [full(a) for a in enc_params],
        out_specs=[
            pl.BlockSpec((rb, 1), lambda i: (i, 0)),
            pl.BlockSpec((1, 1, OUT), lambda i: (i, 0, 0)),
        ],
        out_shape=[
            jax.ShapeDtypeStruct((n, 1), jnp.float32),
            jax.ShapeDtypeStruct((grid[0], 1, OUT), jnp.float32),
        ],
    )(cls2, pos2, xc, xtg, *enc_params)
    return att_raw, pools.reshape(grid[0], OUT)


def kernel(x_cell, centroids_cell, x_tissue_3, centroids_tissue_3, assignment_matrix_3, params):
    p = params
    batch_idx = jnp.argmax(assignment_matrix_3, axis=1)

    ca = _attr_tower(x_cell, p['ca1_w'], p['ca1_b'], p['ca2_w'], p['ca2_b'])
    cl = _bn(centroids_cell, p['cl_bn_g'], p['cl_bn_b'])
    cl = _lin(_lrelu(_lin(cl, p['cl1_w'], p['cl1_b'])), p['cl2_w'], p['cl2_b'])
    cell_feat = jnp.concatenate([ca, cl], axis=1)
    ta = _attr_tower(x_tissue_3, p['ta1_w'], p['ta1_b'], p['ta2_w'], p['ta2_b'])
    tl = _bn(centroids_tissue_3, p['tl_bn_g'], p['tl_bn_b'])
    tl = _lin(_lrelu(_lin(tl, p['tl1_w'], p['tl1_b'])), p['tl2_w'], p['tl2_b'])
    tis_feat = jnp.concatenate([ta, tl], axis=1)

    c_idx, c_valid = _radius_topk(cell_feat, R, rb=1000, ct=512)
    t_idx, t_valid = _radius_topk(tis_feat, R, rb=1000, ct=512)

    xc = x_cell
    for j in (1, 2, 3):
        xc = _sage(xc, c_idx, c_valid, p['c%d_wl' % j], p['c%d_wr' % j], p['c%d_b' % j], N_CELL)
        xc = _lrelu(_graph_norm(xc, p['gn_g'], p['gn_b'], p['gn_a']))
    xt = x_tissue_3
    for j in (4, 5, 6):
        xt = _sage(xt, t_idx, t_valid, p['c%d_wl' % j], p['c%d_wr' % j], p['c%d_b' % j], N_TISSUE)
        xt = _lrelu(_graph_norm(xt, p['gn_g'], p['gn_b'], p['gn_a']))

    xtg = xt[batch_idx]
    att_raw, pools = _encoder_head(xc, xtg, p)
    att = jax.nn.softmax(att_raw, axis=0)
    pooled = jnp.sum(pools, axis=0, keepdims=True) * (1.0 / N_CELL)
    h = _lrelu(_lin(pooled, p['lin1_w'], p['lin1_b']))
    h = _layer_norm(h, p['ln2_g'], p['ln2_b'])
    logits = _lin(h, p['lin2_w'], p['lin2_b'])
    c_src = c_idx.reshape(-1)
    c_dst = jnp.broadcast_to(
        jnp.arange(N_CELL, dtype=c_idx.dtype)[:, None], (N_CELL, K)).reshape(-1)
    edge_index_cell = jnp.stack([c_src, c_dst])
    return logits, edge_index_cell, att
